# baseline (speedup 1.0000x reference)
import jax
import jax.numpy as jnp
from jax.experimental import pallas as pl
from jax.experimental.pallas import tpu as pltpu

_BLK = 512


def _body(ar_ref, gate_ref, x_ref, emb_ref, o_ref):
    b = pl.program_id(0)
    t = pl.program_id(1)
    h = ar_ref[b, 0]
    w = ar_ref[b, 1]
    scale = jnp.where(t < h * w, jnp.tanh(gate_ref[0]), jnp.float32(0.0))
    o_ref[...] = x_ref[...] + emb_ref[...] * scale


def kernel(x, ar, embedding, gate):
    bsz, num_tiles, ntok, width = x.shape
    nblk = (ntok + _BLK - 1) // _BLK

    def x_map(b, t, n, ar_ref, gate_ref):
        return (b, t, n, 0)

    def emb_map(b, t, n, ar_ref, gate_ref):
        w_safe = jnp.maximum(ar_ref[b, 1], 1)
        return (t // w_safe, t % w_safe, 0, 0)

    grid_spec = pltpu.PrefetchScalarGridSpec(
        num_scalar_prefetch=2,
        grid=(bsz, num_tiles, nblk),
        in_specs=[
            pl.BlockSpec((1, 1, _BLK, width), x_map),
            pl.BlockSpec((1, 1, 1, width), emb_map),
        ],
        out_specs=pl.BlockSpec((1, 1, _BLK, width), x_map),
    )
    return pl.pallas_call(
        _body,
        grid_spec=grid_spec,
        out_shape=jax.ShapeDtypeStruct(x.shape, x.dtype),
        compiler_params=pltpu.CompilerParams(
            dimension_semantics=("parallel", "arbitrary", "arbitrary"),
        ),
    )(ar, gate, x, embedding)
